# baseline (speedup 1.0000x reference)
import jax
import jax.numpy as jnp
from jax import lax
from jax.experimental import pallas as pl
from jax.experimental.pallas import tpu as pltpu

D_MODEL = 768
E = 64


def _router_body(x_ref, w_ref, p_ref, i_ref, mask_ref, psum_ref, msum_ref,
                 ent_ref, pacc, macc, eacc):
    T = x_ref.shape[0]
    logits = lax.dot_general(
        w_ref[:], x_ref[:], (((1,), (1,)), ((), ())),
        preferred_element_type=jnp.float32)
    row = lax.broadcasted_iota(jnp.int32, (E, T), 0)

    m = jnp.max(logits, axis=0, keepdims=True)
    e = jnp.exp(logits - m)
    s = jnp.sum(e, axis=0, keepdims=True)
    r = 1.0 / s
    q = jnp.sum(e * (logits - m), axis=0, keepdims=True)

    i1 = jnp.min(jnp.where(logits == m, row, E), axis=0, keepdims=True)
    lm = jnp.where(row == i1, -jnp.inf, logits)
    m2 = jnp.max(lm, axis=0, keepdims=True)
    i2 = jnp.min(jnp.where(lm == m2, row, E), axis=0, keepdims=True)

    hits = ((row == i1) | (row == i2)).astype(jnp.bfloat16)
    eye_e = jnp.eye(E, dtype=jnp.bfloat16)
    mask_ref[:] = lax.dot_general(
        hits, eye_e, (((0,), (0,)), ((), ())),
        preferred_element_type=jnp.float32)

    p_ref[:] = jnp.transpose(jnp.concatenate([r, jnp.exp(m2 - m) / s], axis=0))
    i_ref[:] = jnp.transpose(jnp.concatenate([i1, i2], axis=0))

    @pl.when(pl.program_id(0) == 0)
    def _init():
        pacc[:] = jnp.zeros_like(pacc)
        macc[:] = jnp.zeros_like(macc)
        eacc[:] = jnp.zeros_like(eacc)

    pacc[:] += jnp.sum(e * r, axis=1, keepdims=True)
    macc[:] += jnp.sum(hits.astype(jnp.float32), axis=1, keepdims=True)
    eacc[:] += jnp.sum(jnp.log(s) - q * r).reshape(1, 1)

    @pl.when(pl.program_id(0) == pl.num_programs(0) - 1)
    def _flush():
        psum_ref[:] = pacc[:]
        msum_ref[:] = macc[:]
        ent_ref[:] = eacc[:]


def kernel(x, W, temp):
    B, S, D = x.shape
    N = B * S
    t = jnp.clip(temp, 0.1, 5.0)
    w = W / t
    xf = x.reshape(N, D)
    T = 4096
    grid = N // T

    p_pair, i_pair, mask, psum, msum, ent = pl.pallas_call(
        _router_body,
        grid=(grid,),
        in_specs=[
            pl.BlockSpec((T, D), lambda i: (i, 0)),
            pl.BlockSpec((E, D), lambda i: (0, 0)),
        ],
        out_specs=[
            pl.BlockSpec((T, 2), lambda i: (i, 0)),
            pl.BlockSpec((T, 2), lambda i: (i, 0)),
            pl.BlockSpec((T, E), lambda i: (i, 0)),
            pl.BlockSpec((E, 1), lambda i: (0, 0)),
            pl.BlockSpec((E, 1), lambda i: (0, 0)),
            pl.BlockSpec((1, 1), lambda i: (0, 0)),
        ],
        out_shape=[
            jax.ShapeDtypeStruct((N, 2), jnp.float32),
            jax.ShapeDtypeStruct((N, 2), jnp.int32),
            jax.ShapeDtypeStruct((N, E), jnp.float32),
            jax.ShapeDtypeStruct((E, 1), jnp.float32),
            jax.ShapeDtypeStruct((E, 1), jnp.float32),
            jax.ShapeDtypeStruct((1, 1), jnp.float32),
        ],
        scratch_shapes=[
            pltpu.VMEM((E, 1), jnp.float32),
            pltpu.VMEM((E, 1), jnp.float32),
            pltpu.VMEM((1, 1), jnp.float32),
        ],
    )(xf, w)

    expert_probs = p_pair.reshape(B, S, 2)
    expert_indices = i_pair.reshape(B, S, 2)
    expert_mask = mask.reshape(B, S, E)

    denom = jnp.float32(N)
    importance = psum[:, 0] / denom
    load = msum[:, 0] / (denom + 1e-6)
    aux_load_loss = jnp.sum(importance * load) * E * 0.01
    router_entropy = (ent[0, 0] / denom) * 0.01
    return expert_probs, expert_indices, expert_mask, aux_load_loss, router_entropy
